# baseline (speedup 1.0000x reference)
import functools

import jax
import jax.numpy as jnp
from jax import lax
from jax.experimental import pallas as pl
from jax.experimental.pallas import tpu as pltpu
from jax.experimental.pallas import tpu_sc as plsc

B = 64
S = 1024
D = 768
NW = 32
ROWS_PER_W = B * S // NW
CHUNK = 64
NCHUNK = ROWS_PER_W // CHUNK
LANES = 16

_mesh = plsc.VectorSubcoreMesh(core_axis_name="c", subcore_axis_name="s")


@functools.partial(
    pl.kernel,
    mesh=_mesh,
    out_type=jax.ShapeDtypeStruct((B * S, D), jnp.float32),
    scratch_types=[
        pltpu.VMEM((CHUNK,), jnp.int32),
        pltpu.VMEM((CHUNK, D), jnp.float32),
        pltpu.VMEM((CHUNK, D), jnp.float32),
        pltpu.SemaphoreType.DMA,
    ],
)
def _emb_kernel(ids_hbm, tok_hbm, pos_hbm, out_hbm, idx_v, tok_v, pos_v, sem):
    wid = lax.axis_index("s") * 2 + lax.axis_index("c")
    base = wid * ROWS_PER_W

    def chunk_body(c, carry):
        row0 = base + c * CHUNK
        pltpu.sync_copy(ids_hbm.at[pl.ds(row0, CHUNK)], idx_v)
        gather = pltpu.async_copy(tok_hbm.at[idx_v], tok_v, sem)
        pos0 = lax.rem(row0, S)
        pltpu.sync_copy(pos_hbm.at[pl.ds(pos0, CHUNK)], pos_v)
        gather.wait()

        def row_body(r, rcarry):
            for j in range(D // LANES):
                sl = pl.ds(j * LANES, LANES)
                tok_v[r, sl] = tok_v[r, sl] + pos_v[r, sl]
            return rcarry

        lax.fori_loop(0, CHUNK, row_body, 0)
        pltpu.sync_copy(tok_v, out_hbm.at[pl.ds(row0, CHUNK)])
        return carry

    lax.fori_loop(0, NCHUNK, chunk_body, 0)


def kernel(input_ids, tok_emb, pos_emb):
    ids_flat = input_ids.reshape(B * S).astype(jnp.int32)
    out = _emb_kernel(ids_flat, tok_emb, pos_emb)
    return out.reshape(B, S, D)


# speedup vs baseline: 1.8774x; 1.8774x over previous
import functools

import jax
import jax.numpy as jnp
from jax import lax
from jax.experimental import pallas as pl
from jax.experimental.pallas import tpu as pltpu
from jax.experimental.pallas import tpu_sc as plsc

B = 64
S = 1024
D = 768
NW = 32
PB = S // NW
LANES = 16
NBUF = 4
NK = B // NBUF

_mesh = plsc.VectorSubcoreMesh(core_axis_name="c", subcore_axis_name="s")


@functools.partial(
    pl.kernel,
    mesh=_mesh,
    out_type=jax.ShapeDtypeStruct((B * S, D), jnp.float32),
    scratch_types=[
        pltpu.VMEM((B, PB), jnp.int32),
        pltpu.VMEM((PB, D), jnp.float32),
        pltpu.VMEM((PB, D), jnp.float32),
        pltpu.VMEM((PB, D), jnp.float32),
        pltpu.VMEM((PB, D), jnp.float32),
        pltpu.VMEM((PB, D), jnp.float32),
        pltpu.SemaphoreType.DMA,
        pltpu.SemaphoreType.DMA,
        pltpu.SemaphoreType.DMA,
        pltpu.SemaphoreType.DMA,
        pltpu.SemaphoreType.DMA,
        pltpu.SemaphoreType.DMA,
        pltpu.SemaphoreType.DMA,
        pltpu.SemaphoreType.DMA,
    ],
)
def _emb_kernel(ids_hbm, tok_hbm, pos_hbm, out_hbm,
                idx_v, pos_v, t0, t1, t2, t3,
                g0, g1, g2, g3, o0, o1, o2, o3):
    toks = (t0, t1, t2, t3)
    gsems = (g0, g1, g2, g3)
    osems = (o0, o1, o2, o3)
    wid = lax.axis_index("s") * 2 + lax.axis_index("c")
    s0 = wid * PB

    def idx_issue(b, carry):
        pltpu.async_copy(ids_hbm.at[pl.ds(b * S + s0, PB)], idx_v.at[b], g0)
        return carry

    def idx_drain(b, carry):
        pltpu.make_async_copy(ids_hbm.at[pl.ds(b * S + s0, PB)], idx_v.at[b],
                              g0).wait()
        return carry

    lax.fori_loop(0, B, idx_issue, 0)
    pltpu.sync_copy(pos_hbm.at[pl.ds(s0, PB)], pos_v)
    lax.fori_loop(0, B, idx_drain, 0)
    for x in range(NBUF):
        pltpu.async_copy(tok_hbm.at[idx_v.at[x]], toks[x], gsems[x])

    def add_block(buf):
        def row_body(r, carry):
            for j in range(D // LANES):
                sl = pl.ds(j * LANES, LANES)
                buf[r, sl] = buf[r, sl] + pos_v[r, sl]
            return carry
        lax.fori_loop(0, PB, row_body, 0)

    def out_slice(b):
        return out_hbm.at[pl.ds(b * S + s0, PB)]

    def k_body(k, carry):
        for j in range(NBUF):
            b = k * NBUF + j
            x = j
            z = (j - 1) % NBUF

            def refill():
                pltpu.make_async_copy(toks[z], out_slice(b - 1), osems[z]).wait()
                nb = b + NBUF - 1
                pltpu.async_copy(tok_hbm.at[idx_v.at[nb]], toks[z], gsems[z])

            if j == 0:
                pl.when(k > 0)(refill)
            else:
                pl.when(k < NK - 1)(refill)

            pltpu.make_async_copy(tok_hbm.at[idx_v.at[b]], toks[x], gsems[x]).wait()
            add_block(toks[x])
            pltpu.async_copy(toks[x], out_slice(b), osems[x])
        return carry

    lax.fori_loop(0, NK, k_body, 0)

    for x in range(NBUF):
        pltpu.make_async_copy(toks[x], out_slice(B - NBUF + x), osems[x]).wait()


def kernel(input_ids, tok_emb, pos_emb):
    ids = input_ids.reshape(B * S).astype(jnp.int32)
    out = _emb_kernel(ids, tok_emb, pos_emb)
    return out.reshape(B, S, D)


# speedup vs baseline: 2.3032x; 1.2268x over previous
import functools

import jax
import jax.numpy as jnp
from jax import lax
from jax.experimental import pallas as pl
from jax.experimental.pallas import tpu as pltpu
from jax.experimental.pallas import tpu_sc as plsc

B = 64
S = 1024
D = 768
NW = 32
PB = S // NW
LANES = 16
NBUF = 4
NK = B // NBUF

_mesh = plsc.VectorSubcoreMesh(core_axis_name="c", subcore_axis_name="s")


@functools.partial(
    pl.kernel,
    mesh=_mesh,
    out_type=jax.ShapeDtypeStruct((B * S, D), jnp.float32),
    scratch_types=[
        pltpu.VMEM((B, PB), jnp.int32),
        pltpu.VMEM((PB, D), jnp.float32),
        pltpu.VMEM((PB, D), jnp.float32),
        pltpu.VMEM((PB, D), jnp.float32),
        pltpu.VMEM((PB, D), jnp.float32),
        pltpu.VMEM((PB, D), jnp.float32),
        pltpu.SemaphoreType.DMA,
        pltpu.SemaphoreType.DMA,
        pltpu.SemaphoreType.DMA,
        pltpu.SemaphoreType.DMA,
        pltpu.SemaphoreType.DMA,
        pltpu.SemaphoreType.DMA,
        pltpu.SemaphoreType.DMA,
        pltpu.SemaphoreType.DMA,
    ],
)
def _emb_kernel(ids_hbm, tok_hbm, pos_hbm, out_hbm,
                idx_v, pos_v, t0, t1, t2, t3,
                g0, g1, g2, g3, o0, o1, o2, o3):
    toks = (t0, t1, t2, t3)
    gsems = (g0, g1, g2, g3)
    osems = (o0, o1, o2, o3)
    wid = lax.axis_index("s") * 2 + lax.axis_index("c")
    s0 = wid * PB

    def idx_issue(b, carry):
        pltpu.async_copy(ids_hbm.at[pl.ds(b * S + s0, PB)], idx_v.at[b], g0)
        return carry

    def idx_drain(b, carry):
        pltpu.make_async_copy(ids_hbm.at[pl.ds(b * S + s0, PB)], idx_v.at[b],
                              g0).wait()
        return carry

    lax.fori_loop(0, B, idx_issue, 0)
    pltpu.sync_copy(pos_hbm.at[pl.ds(s0, PB)], pos_v)
    lax.fori_loop(0, B, idx_drain, 0)
    for x in range(2):
        pltpu.async_copy(tok_hbm.at[idx_v.at[x]], toks[x], gsems[x])

    def add_block(buf):
        def row_body(r, carry):
            for j in range(D // LANES):
                sl = pl.ds(j * LANES, LANES)
                buf[r, sl] = buf[r, sl] + pos_v[r, sl]
            return carry
        lax.fori_loop(0, PB, row_body, 0)

    def out_slice(b):
        return out_hbm.at[pl.ds(b * S + s0, PB)]

    def k_body(k, carry):
        for j in range(NBUF):
            b = k * NBUF + j
            x = j
            z = (j + 2) % NBUF

            def refill_wait():
                pltpu.make_async_copy(toks[z], out_slice(b - 2), osems[z]).wait()

            def refill_issue():
                pltpu.async_copy(tok_hbm.at[idx_v.at[b + 2]], toks[z], gsems[z])

            if j < 2:
                pl.when(k > 0)(refill_wait)
                refill_issue()
            else:
                refill_wait()
                pl.when(k < NK - 1)(refill_issue)

            pltpu.make_async_copy(tok_hbm.at[idx_v.at[b]], toks[x], gsems[x]).wait()
            add_block(toks[x])
            pltpu.async_copy(toks[x], out_slice(b), osems[x])
        return carry

    lax.fori_loop(0, NK, k_body, 0)

    for x in (2, 3):
        pltpu.make_async_copy(toks[x], out_slice(B - 4 + x), osems[x]).wait()


def kernel(input_ids, tok_emb, pos_emb):
    ids = input_ids.reshape(B * S).astype(jnp.int32)
    out = _emb_kernel(ids, tok_emb, pos_emb)
    return out.reshape(B, S, D)


# speedup vs baseline: 2.3034x; 1.0001x over previous
import functools

import jax
import jax.numpy as jnp
from jax import lax
from jax.experimental import pallas as pl
from jax.experimental.pallas import tpu as pltpu
from jax.experimental.pallas import tpu_sc as plsc

B = 64
S = 1024
D = 768
NW = 32
PB = S // NW
LANES = 16
NBUF = 4
NK = B // NBUF

_mesh = plsc.VectorSubcoreMesh(core_axis_name="c", subcore_axis_name="s")


@functools.partial(
    pl.kernel,
    mesh=_mesh,
    out_type=jax.ShapeDtypeStruct((B * S, D), jnp.float32),
    scratch_types=[
        pltpu.VMEM((B, PB), jnp.int32),
        pltpu.VMEM((PB, D), jnp.float32),
        pltpu.VMEM((PB, D), jnp.float32),
        pltpu.VMEM((PB, D), jnp.float32),
        pltpu.VMEM((PB, D), jnp.float32),
        pltpu.VMEM((PB, D), jnp.float32),
        pltpu.SemaphoreType.DMA,
        pltpu.SemaphoreType.DMA,
        pltpu.SemaphoreType.DMA,
        pltpu.SemaphoreType.DMA,
        pltpu.SemaphoreType.DMA,
        pltpu.SemaphoreType.DMA,
        pltpu.SemaphoreType.DMA,
        pltpu.SemaphoreType.DMA,
    ],
)
def _emb_kernel(ids_hbm, tok_hbm, pos_hbm, out_hbm,
                idx_v, pos_v, t0, t1, t2, t3,
                g0, g1, g2, g3, o0, o1, o2, o3):
    toks = (t0, t1, t2, t3)
    gsems = (g0, g1, g2, g3)
    osems = (o0, o1, o2, o3)
    wid = lax.axis_index("s") * 2 + lax.axis_index("c")
    s0 = wid * PB

    def idx_issue(b, carry):
        pltpu.async_copy(ids_hbm.at[pl.ds(b * S + s0, PB)], idx_v.at[b], g0)
        return carry

    def idx_drain(b, carry):
        pltpu.make_async_copy(ids_hbm.at[pl.ds(b * S + s0, PB)], idx_v.at[b],
                              g0).wait()
        return carry

    lax.fori_loop(0, B, idx_issue, 0)
    pltpu.sync_copy(pos_hbm.at[pl.ds(s0, PB)], pos_v)
    lax.fori_loop(0, B, idx_drain, 0)
    for x in range(2):
        pltpu.async_copy(tok_hbm.at[idx_v.at[x]], toks[x], gsems[x])

    def add_block(buf):
        def row_body(r, carry):
            for rr in range(2):
                for j in range(D // LANES):
                    sl = pl.ds(j * LANES, LANES)
                    plsc.addupdate(buf.at[2 * r + rr, sl], pos_v[2 * r + rr, sl])
            return carry
        lax.fori_loop(0, PB // 2, row_body, 0)

    def out_slice(b):
        return out_hbm.at[pl.ds(b * S + s0, PB)]

    def k_body(k, carry):
        for j in range(NBUF):
            b = k * NBUF + j
            x = j
            z = (j + 2) % NBUF

            def refill_wait():
                pltpu.make_async_copy(toks[z], out_slice(b - 2), osems[z]).wait()

            def refill_issue():
                pltpu.async_copy(tok_hbm.at[idx_v.at[b + 2]], toks[z], gsems[z])

            if j < 2:
                pl.when(k > 0)(refill_wait)
                refill_issue()
            else:
                refill_wait()
                pl.when(k < NK - 1)(refill_issue)

            pltpu.make_async_copy(tok_hbm.at[idx_v.at[b]], toks[x], gsems[x]).wait()
            add_block(toks[x])
            pltpu.async_copy(toks[x], out_slice(b), osems[x])
        return carry

    lax.fori_loop(0, NK, k_body, 0)

    for x in (2, 3):
        pltpu.make_async_copy(toks[x], out_slice(B - 4 + x), osems[x]).wait()


def kernel(input_ids, tok_emb, pos_emb):
    ids = input_ids.reshape(B * S).astype(jnp.int32)
    out = _emb_kernel(ids, tok_emb, pos_emb)
    return out.reshape(B, S, D)
